# Initial kernel scaffold; baseline (speedup 1.0000x reference)
import jax
import jax.numpy as jnp
from jax.experimental import pallas as pl


def kernel(x, edge_index, W1, b1, W_mu, b_mu, W_lv, b_lv):
    raise NotImplementedError("write your pallas kernel here")


# speedup vs baseline: 41.8689x; 41.8689x over previous
import functools

import jax
import jax.numpy as jnp
from jax import lax
from jax.experimental import pallas as pl
from jax.experimental.pallas import tpu as pltpu
from jax.experimental.pallas import tpu_sc as plsc

_N = 10000
_E = 320000
_IN_CH = 128
_HID = 32
_LAT = 16

_NC = 2
_NS = 16
_NW = _NC * _NS
_CHUNK = 128
_K = 79
_EPW = _K * _CHUNK
_E_PAD = _NW * _EPW
_N_PAD = 10240
_RPT = _N_PAD // _NS

_mesh = plsc.VectorSubcoreMesh(core_axis_name="c", subcore_axis_name="s")
_sc_params = pltpu.CompilerParams(needs_layout_passes=False,
                                  use_tc_tiling_on_sc=False)


@functools.partial(
    pl.kernel,
    out_type=jax.ShapeDtypeStruct((_NW, _N_PAD), jnp.float32),
    mesh=_mesh,
    scratch_types=[
        pltpu.VMEM((_EPW,), jnp.int32),
        pltpu.VMEM((_N_PAD,), jnp.float32),
    ],
    compiler_params=_sc_params,
)
def _sc_degree(dst_flat_hbm, out_hbm, dstv, deg):
    wid = lax.axis_index("c") * _NS + lax.axis_index("s")

    def _zero(i, _):
        deg[pl.ds(i * 16, 16)] = jnp.zeros((16,), jnp.float32)
        return 0

    lax.fori_loop(0, _N_PAD // 16, _zero, 0)
    pltpu.sync_copy(dst_flat_hbm.at[wid], dstv)
    ones = jnp.ones((16,), jnp.float32)

    def _acc(v, _):
        idx = dstv[pl.ds(v * 16, 16)]
        plsc.addupdate_scatter(deg, [idx], ones)
        return 0

    lax.fori_loop(0, _EPW // 16, _acc, 0)
    pltpu.sync_copy(deg, out_hbm.at[wid])


@functools.partial(
    pl.kernel,
    out_type=jax.ShapeDtypeStruct((_NC, _N_PAD, _HID), jnp.float32),
    mesh=_mesh,
    scratch_types=[
        pltpu.VMEM((_K, _CHUNK), jnp.int32),
        pltpu.VMEM((_K, _CHUNK), jnp.int32),
        pltpu.VMEM((_CHUNK, _HID), jnp.float32),
        pltpu.VMEM((_CHUNK, _HID), jnp.float32),
        pltpu.SemaphoreType.DMA,
        pltpu.SemaphoreType.DMA,
        pltpu.VMEM_SHARED((_N_PAD, _HID), jnp.float32),
    ],
    compiler_params=_sc_params,
)
def _sc_scatter_rows(table_hbm, src_hbm, dst_hbm, zeros_hbm, out_hbm,
                     srcv, dstv, rows0, rows1, sem0, sem1, acc):
    c = lax.axis_index("c")
    s = lax.axis_index("s")
    wid = c * _NS + s
    r0 = s * _RPT

    pltpu.sync_copy(zeros_hbm.at[pl.ds(r0, _RPT)], acc.at[pl.ds(r0, _RPT)])
    pltpu.sync_copy(src_hbm.at[wid], srcv)
    pltpu.sync_copy(dst_hbm.at[wid], dstv)
    plsc.subcore_barrier()

    cp0 = pltpu.async_copy(table_hbm.at[srcv.at[0]], rows0, sem0)

    def _step(j, _):
        even = j % 2 == 0

        def _go(cur, nxt, cur_sem, nxt_sem):
            nc = pltpu.async_copy(table_hbm.at[srcv.at[j + 1]], nxt, nxt_sem)
            pltpu.make_async_copy(table_hbm.at[srcv.at[j]], cur, cur_sem).wait()
            pltpu.sync_copy(cur, acc.at[dstv.at[j]], add=True)
            return nc

        @pl.when(even)
        def _():
            _go(rows0, rows1, sem0, sem1)

        @pl.when(jnp.logical_not(even))
        def _():
            _go(rows1, rows0, sem1, sem0)

        return 0

    lax.fori_loop(0, _K - 1, _step, 0)
    last = _K - 1
    buf, sem = (rows0, sem0) if last % 2 == 0 else (rows1, sem1)
    pltpu.make_async_copy(table_hbm.at[srcv.at[last]], buf, sem).wait()
    pltpu.sync_copy(buf, acc.at[dstv.at[last]], add=True)

    plsc.subcore_barrier()
    pltpu.sync_copy(acc.at[pl.ds(r0, _RPT)], out_hbm.at[c, pl.ds(r0, _RPT)])


def _tc_prepare1_body(deg_parts, x, w1, table1, dinv):
    deg = 1.0 + jnp.sum(deg_parts[...], axis=0)
    di = lax.rsqrt(deg)
    h = jnp.dot(x[...], w1[...], preferred_element_type=jnp.float32)
    table1[...] = h * di[:, None]
    dinv[...] = di[:, None]


def _tc_prepare2_body(accs, table1, dinv, b1, wcat, table2):
    di = dinv[...]
    pre = (accs[0] + accs[1] + table1[...]) * di + b1[...][None, :]
    h = jnp.maximum(pre, 0.0)
    table2[...] = jnp.dot(h, wcat[...], preferred_element_type=jnp.float32) * di


def _tc_final_body(accs, table2, dinv, bcat, out):
    out[...] = (accs[0] + accs[1] + table2[...]) * dinv[...] + bcat[...][None, :]


def kernel(x, edge_index, W1, b1, W_mu, b_mu, W_lv, b_lv):
    src = edge_index[0]
    dst = edge_index[1]
    pad = _E_PAD - _E
    src_p = jnp.concatenate([src, jnp.full((pad,), _N, jnp.int32)])
    dst_p = jnp.concatenate([dst, jnp.full((pad,), _N, jnp.int32)])
    src_w = src_p.reshape(_NW, _K, _CHUNK)
    dst_w = dst_p.reshape(_NW, _K, _CHUNK)
    dst_flat = dst_p.reshape(_NW, _EPW)

    x_p = jnp.zeros((_N_PAD, _IN_CH), jnp.float32).at[:_N].set(x)
    wcat = jnp.concatenate([W_mu, W_lv], axis=1)
    bcat = jnp.concatenate([b_mu, b_lv])
    zeros_rows = jnp.zeros((_N_PAD, _HID), jnp.float32)

    deg_parts = _sc_degree(dst_flat)

    table1, dinv = pl.pallas_call(
        _tc_prepare1_body,
        out_shape=(
            jax.ShapeDtypeStruct((_N_PAD, _HID), jnp.float32),
            jax.ShapeDtypeStruct((_N_PAD, 1), jnp.float32),
        ),
    )(deg_parts, x_p, W1)

    acc1 = _sc_scatter_rows(table1, src_w, dst_w, zeros_rows)

    table2 = pl.pallas_call(
        _tc_prepare2_body,
        out_shape=jax.ShapeDtypeStruct((_N_PAD, _HID), jnp.float32),
    )(acc1, table1, dinv, b1, wcat)

    acc2 = _sc_scatter_rows(table2, src_w, dst_w, zeros_rows)

    out = pl.pallas_call(
        _tc_final_body,
        out_shape=jax.ShapeDtypeStruct((_N_PAD, _HID), jnp.float32),
    )(acc2, table2, dinv, bcat)

    return (out[:_N, :_LAT], out[:_N, _LAT:])
